# Initial kernel scaffold; baseline (speedup 1.0000x reference)
import jax
import jax.numpy as jnp
from jax.experimental import pallas as pl


def kernel(X, Nbrs, Nbrs_Z):
    raise NotImplementedError("write your pallas kernel here")


# speedup vs baseline: 137.5680x; 137.5680x over previous
import functools
import math

import jax
import jax.numpy as jnp
from jax import lax
from jax.experimental import pallas as pl
from jax.experimental.pallas import tpu as pltpu
from jax.experimental.pallas import tpu_sc as plsc

B, N, M, D_FEAT = 32, 4096, 32, 3
NPAR = 8
LANES = 16
CH = 1024
RC = 12.0
ETA = 0.04
RS_STEP = 1.5
G_COEF = 2.0 * ETA * RS_STEP
C_K = [math.exp(-ETA * (RS_STEP * k) ** 2) for k in range(NPAR)]


def _sc_layer(xt, nbrs, nbrs_z):
    mesh = plsc.VectorSubcoreMesh(core_axis_name="c", subcore_axis_name="s")

    @functools.partial(
        pl.kernel,
        mesh=mesh,
        compiler_params=pltpu.CompilerParams(needs_layout_passes=False),
        out_type=jax.ShapeDtypeStruct((B, N * NPAR), jnp.float32),
        scratch_types=[
            pltpu.VMEM((N,), jnp.float32),
            pltpu.VMEM((N,), jnp.float32),
            pltpu.VMEM((N,), jnp.float32),
            pltpu.VMEM((CH * M,), jnp.int32),
            pltpu.VMEM((CH * M,), jnp.int32),
            pltpu.VMEM((CH * NPAR,), jnp.float32),
        ],
    )
    def k(xt_hbm, nb_hbm, z_hbm, out_hbm, xp, yp, zp, nb_v, z_v, stage):
        b = lax.axis_index("s") * 2 + lax.axis_index("c")
        pltpu.sync_copy(xt_hbm.at[3 * b + 0], xp)
        pltpu.sync_copy(xt_hbm.at[3 * b + 1], yp)
        pltpu.sync_copy(xt_hbm.at[3 * b + 2], zp)

        lanes = lax.iota(jnp.int32, LANES)
        lane_m = lanes * M
        lane_p = lanes * NPAR

        def chunk_body(c, carry):
            off = c * (CH * M)
            pltpu.sync_copy(nb_hbm.at[b, pl.ds(off, CH * M)], nb_v)
            pltpu.sync_copy(z_hbm.at[b, pl.ds(off, CH * M)], z_v)

            def group_body(g, carry2):
                a0 = c * CH + g * LANES
                own_idx = lanes + a0
                xo = plsc.load_gather(xp, [own_idx])
                yo = plsc.load_gather(yp, [own_idx])
                zo = plsc.load_gather(zp, [own_idx])
                gbase = g * (LANES * M)

                def m_body(m, accs):
                    idx = lane_m + (gbase + m)
                    ni = plsc.load_gather(nb_v, [idx])
                    zz = plsc.load_gather(z_v, [idx])
                    xn = plsc.load_gather(xp, [ni])
                    yn = plsc.load_gather(yp, [ni])
                    zn = plsc.load_gather(zp, [ni])
                    dx = xn - xo
                    dy = yn - yo
                    dz = zn - zo
                    s = dx * dx + dy * dy + dz * dz
                    i = lax.bitcast_convert_type(s, jnp.int32)
                    i = 0x5F3759DF - lax.shift_right_arithmetic(i, 1)
                    y = lax.bitcast_convert_type(i, jnp.float32)
                    hs = 0.5 * s
                    y = y * (1.5 - hs * y * y)
                    y = y * (1.5 - hs * y * y)
                    y = y * (1.5 - hs * y * y)
                    r = s * y
                    rcl = jnp.minimum(r, RC)
                    u = rcl * (math.pi / RC) - (0.5 * math.pi)
                    u2 = u * u
                    sinu = u * (1.0 + u2 * (-1.0 / 6 + u2 * (1.0 / 120
                                + u2 * (-1.0 / 5040 + u2 * (1.0 / 362880)))))
                    fc = 0.5 - 0.5 * sinu
                    ok = jnp.logical_and(r <= RC, zz != 0)
                    w = jnp.where(ok, fc, 0.0)
                    a = jnp.exp(-ETA * (rcl * rcl))
                    gg = jnp.exp(G_COEF * rcl)
                    t = w * a
                    new = []
                    for kk in range(NPAR):
                        new.append(accs[kk] + t * C_K[kk])
                        if kk < NPAR - 1:
                            t = t * gg
                    return tuple(new)

                zero = jnp.zeros((LANES,), jnp.float32)
                accs = lax.fori_loop(0, M, m_body, (zero,) * NPAR)
                sbase = g * (LANES * NPAR)
                for kk in range(NPAR):
                    plsc.store_scatter(stage, [lane_p + (sbase + kk)], accs[kk])
                return carry2

            lax.fori_loop(0, CH // LANES, group_body, 0)
            pltpu.sync_copy(stage, out_hbm.at[b, pl.ds(c * (CH * NPAR), CH * NPAR)])
            return carry

        lax.fori_loop(0, N // CH, chunk_body, 0)

    return k(xt, nbrs, nbrs_z)


def _bn(layer):
    cols = N * NPAR // 16

    def body(x_ref, o_ref):
        x = x_ref[...]
        mu = jnp.mean(x, axis=0, keepdims=True)
        d = x - mu
        var = jnp.mean(d * d, axis=0, keepdims=True)
        o_ref[...] = d * lax.rsqrt(var + 0.001)

    return pl.pallas_call(
        body,
        grid=(16,),
        in_specs=[pl.BlockSpec((B, cols), lambda i: (0, i))],
        out_specs=pl.BlockSpec((B, cols), lambda i: (0, i)),
        out_shape=jax.ShapeDtypeStruct((B, N * NPAR), jnp.float32),
    )(layer)


def kernel(X, Nbrs, Nbrs_Z):
    xt = jnp.transpose(X, (0, 2, 1)).reshape(B * D_FEAT, N)
    nb = Nbrs.reshape(B, N * M)
    zf = Nbrs_Z.reshape(B, N * M)
    layer = _sc_layer(xt, nb, zf)
    out = _bn(layer)
    return out.reshape(B, N, NPAR)


# speedup vs baseline: 145.9843x; 1.0612x over previous
import functools
import math

import jax
import jax.numpy as jnp
from jax import lax
from jax.experimental import pallas as pl
from jax.experimental.pallas import tpu as pltpu
from jax.experimental.pallas import tpu_sc as plsc

B, N, M, D_FEAT = 32, 4096, 32, 3
NPAR = 8
LANES = 16
CH = 1024
RC = 12.0
ETA = 0.04
RS_STEP = 1.5
G_COEF = 2.0 * ETA * RS_STEP
C_K = [math.exp(-ETA * (RS_STEP * k) ** 2) for k in range(NPAR)]


def _sc_layer(xt, nbrs, nbrs_z):
    mesh = plsc.VectorSubcoreMesh(core_axis_name="c", subcore_axis_name="s")

    @functools.partial(
        pl.kernel,
        mesh=mesh,
        compiler_params=pltpu.CompilerParams(needs_layout_passes=False),
        out_type=jax.ShapeDtypeStruct((B, N * NPAR), jnp.float32),
        scratch_types=[
            pltpu.VMEM((N,), jnp.float32),
            pltpu.VMEM((N,), jnp.float32),
            pltpu.VMEM((N,), jnp.float32),
            pltpu.VMEM((CH * M,), jnp.int32),
            pltpu.VMEM((CH * M,), jnp.int32),
            pltpu.VMEM((CH * NPAR,), jnp.float32),
        ],
    )
    def k(xt_hbm, nb_hbm, z_hbm, out_hbm, xp, yp, zp, nb_v, z_v, stage):
        b = lax.axis_index("s") * 2 + lax.axis_index("c")
        pltpu.sync_copy(xt_hbm.at[3 * b + 0], xp)
        pltpu.sync_copy(xt_hbm.at[3 * b + 1], yp)
        pltpu.sync_copy(xt_hbm.at[3 * b + 2], zp)

        lanes = lax.iota(jnp.int32, LANES)
        lane_m = lanes * M
        lane_p = lanes * NPAR

        def chunk_body(c, carry):
            off = c * (CH * M)
            pltpu.sync_copy(nb_hbm.at[b, pl.ds(off, CH * M)], nb_v)
            pltpu.sync_copy(z_hbm.at[b, pl.ds(off, CH * M)], z_v)

            def group_body(g, carry2):
                a0 = c * CH + g * LANES
                xo = xp[pl.ds(a0, LANES)]
                yo = yp[pl.ds(a0, LANES)]
                zo = zp[pl.ds(a0, LANES)]
                gbase = g * (LANES * M)

                def pair(idx, accs):
                    ni = plsc.load_gather(nb_v, [idx])
                    zz = plsc.load_gather(z_v, [idx])
                    xn = plsc.load_gather(xp, [ni])
                    yn = plsc.load_gather(yp, [ni])
                    zn = plsc.load_gather(zp, [ni])
                    dx = xn - xo
                    dy = yn - yo
                    dz = zn - zo
                    s = dx * dx + dy * dy + dz * dz
                    i = lax.bitcast_convert_type(s, jnp.int32)
                    i = 0x5F3759DF - lax.shift_right_arithmetic(i, 1)
                    y = lax.bitcast_convert_type(i, jnp.float32)
                    hs = 0.5 * s
                    y = y * (1.5 - hs * y * y)
                    y = y * (1.5 - hs * y * y)
                    r = s * y
                    rcl = jnp.minimum(r, RC)
                    u = rcl * (math.pi / RC) - (0.5 * math.pi)
                    u2 = u * u
                    hsinu = u * (0.5 + u2 * (-0.5 / 6 + u2 * (0.5 / 120
                                + u2 * (-0.5 / 5040 + u2 * (0.5 / 362880)))))
                    fc = 0.5 - hsinu
                    ok = jnp.logical_and(r <= RC, zz != 0)
                    w = jnp.where(ok, fc, 0.0)
                    a = jnp.exp(-ETA * (rcl * rcl))
                    gg = jnp.exp(G_COEF * rcl)
                    t = w * a
                    new = []
                    for kk in range(NPAR):
                        new.append(accs[kk] + t * C_K[kk])
                        if kk < NPAR - 1:
                            t = t * gg
                    return tuple(new)

                UNROLL = 4

                def m_body(mm, accs):
                    base = gbase + mm * UNROLL
                    for j in range(UNROLL):
                        accs = pair(lane_m + (base + j), accs)
                    return accs

                zero = jnp.zeros((LANES,), jnp.float32)
                accs = lax.fori_loop(0, M // UNROLL, m_body, (zero,) * NPAR)
                sbase = g * (LANES * NPAR)
                for kk in range(NPAR):
                    plsc.store_scatter(stage, [lane_p + (sbase + kk)], accs[kk])
                return carry2

            lax.fori_loop(0, CH // LANES, group_body, 0)
            pltpu.sync_copy(stage, out_hbm.at[b, pl.ds(c * (CH * NPAR), CH * NPAR)])
            return carry

        lax.fori_loop(0, N // CH, chunk_body, 0)

    return k(xt, nbrs, nbrs_z)


def _bn(layer):
    cols = N * NPAR // 16

    def body(x_ref, o_ref):
        x = x_ref[...]
        mu = jnp.mean(x, axis=0, keepdims=True)
        d = x - mu
        var = jnp.mean(d * d, axis=0, keepdims=True)
        o_ref[...] = d * lax.rsqrt(var + 0.001)

    return pl.pallas_call(
        body,
        grid=(16,),
        in_specs=[pl.BlockSpec((B, cols), lambda i: (0, i))],
        out_specs=pl.BlockSpec((B, cols), lambda i: (0, i)),
        out_shape=jax.ShapeDtypeStruct((B, N * NPAR), jnp.float32),
    )(layer)


def kernel(X, Nbrs, Nbrs_Z):
    xt = jnp.transpose(X, (0, 2, 1)).reshape(B * D_FEAT, N)
    nb = Nbrs.reshape(B, N * M)
    zf = Nbrs_Z.reshape(B, N * M)
    layer = _sc_layer(xt, nb, zf)
    out = _bn(layer)
    return out.reshape(B, N, NPAR)
